# baseline (speedup 1.0000x reference)
import functools

import jax
import jax.numpy as jnp
from jax import lax
from jax.experimental import pallas as pl
from jax.experimental.pallas import tpu as pltpu
from jax.experimental.pallas import tpu_sc as plsc

NUM_CORES = 2
NUM_SUBCORES = 16
LANES = 16
NUM_WORKERS = NUM_CORES * NUM_SUBCORES

BATCH = 16384
EMBED_DIM = 32
SUPER = 4
SUPER_DIM = SUPER * EMBED_DIM
B_PER_W = BATCH // NUM_WORKERS
REQ_PER_W = 2 * B_PER_W
CHUNK = 256
NCHUNK = REQ_PER_W // CHUNK
B_PER_CHUNK = CHUNK // 2
GROUPS_PER_CHUNK = B_PER_CHUNK // LANES

_mesh = plsc.VectorSubcoreMesh(core_axis_name="c", subcore_axis_name="s")


@functools.partial(
    pl.kernel,
    out_type=jax.ShapeDtypeStruct((BATCH,), jnp.float32),
    mesh=_mesh,
    scratch_types=[
        pltpu.VMEM((REQ_PER_W,), jnp.int32),
        pltpu.VMEM((REQ_PER_W,), jnp.int32),
        pltpu.VMEM((CHUNK, SUPER_DIM), jnp.float32),
        pltpu.VMEM((CHUNK, SUPER_DIM), jnp.float32),
        pltpu.VMEM((B_PER_W,), jnp.float32),
        pltpu.SemaphoreType.DMA,
        pltpu.SemaphoreType.DMA,
    ],
    compiler_params=pltpu.CompilerParams(
        needs_layout_passes=False, use_tc_tiling_on_sc=True
    ),
)
def _road2vec_sc(xflat_hbm, tsup_hbm, out_hbm,
                 xidx_v, sidx_v, buf0_v, buf1_v, out_v, sem0, sem1):
    wid = lax.axis_index("s") * NUM_CORES + lax.axis_index("c")
    rbase = wid * REQ_PER_W
    obase = wid * B_PER_W

    pltpu.sync_copy(xflat_hbm.at[pl.ds(rbase, REQ_PER_W)], xidx_v)
    for i in range(REQ_PER_W // LANES):
        sidx_v[pl.ds(i * LANES, LANES)] = (
            lax.shift_right_logical(xidx_v[pl.ds(i * LANES, LANES)], 2)
        )

    bufs = (buf0_v, buf1_v)
    sems = (sem0, sem1)

    def start_gather(c):
        return pltpu.async_copy(
            tsup_hbm.at[sidx_v.at[pl.ds(c * CHUNK, CHUNK)]],
            bufs[c % 2], sems[c % 2],
        )

    lane = lax.iota(jnp.int32, LANES)
    copies = [None, None]
    copies[0] = start_gather(0)

    for c in range(NCHUNK):
        if c + 1 < NCHUNK:
            copies[(c + 1) % 2] = start_gather(c + 1)
        copies[c % 2].wait()
        buf = bufs[c % 2]
        cbase = c * CHUNK

        def group(g, carry, buf=buf, cbase=cbase):
            j0 = g * (2 * LANES) + 2 * lane
            j1 = j0 + 1
            x0 = plsc.load_gather(xidx_v, [cbase + j0])
            x1 = plsc.load_gather(xidx_v, [cbase + j1])
            o0 = lax.shift_left(x0 & 3, 5)
            o1 = lax.shift_left(x1 & 3, 5)
            acc = jnp.zeros((LANES,), jnp.float32)
            for d in range(EMBED_DIM):
                u = plsc.load_gather(buf, [j0, o0 + d])
                v = plsc.load_gather(buf, [j1, o1 + d])
                acc = acc + u * v
            out_v[pl.ds(cbase // 2 + g * LANES, LANES)] = (
                1.0 / (1.0 + jnp.exp(-acc))
            )
            return carry

        lax.fori_loop(0, GROUPS_PER_CHUNK, group, 0)

    pltpu.sync_copy(out_v, out_hbm.at[pl.ds(obase, B_PER_W)])


def kernel(x, table):
    xflat = x.reshape(-1).astype(jnp.int32)
    tsup = table.reshape(-1, SUPER_DIM)
    return _road2vec_sc(xflat, tsup)
